# Initial kernel scaffold; baseline (speedup 1.0000x reference)
import jax
import jax.numpy as jnp
from jax.experimental import pallas as pl


def kernel(x, edge_index, W_src1, W_dst1, att_src1, att_dst1, bg1, W_src2, W_dst2, att_src2, att_dst2, bg2, Wf1, bf1, g1, be1, Wf2, bf2, g2, be2, Wf3, bf3, g3, be3, Wf4, bf4):
    raise NotImplementedError("write your pallas kernel here")


# speedup vs baseline: 8.1812x; 8.1812x over previous
import functools

import jax
import jax.numpy as jnp
from jax import lax
from jax.experimental import pallas as pl
from jax.experimental.pallas import tpu as pltpu
from jax.experimental.pallas import tpu_sc as plsc

N = 10000
NP = 10240
E = 320000
ET = E + N
NW = 32
B = 16
NCH = 16
CT = -(-ET // (NCH * B))
EP = NCH * B * CT
EPS = 1e-5

_GDN = lax.GatherDimensionNumbers(offset_dims=(), collapsed_slice_dims=(0,),
                                  start_index_map=(0,))


def _permute(v, idx16):
    return lax.gather(v, idx16[:, None], _GDN, (1,),
                      mode=lax.GatherScatterMode.PROMISE_IN_BOUNDS)


def _edge_body(src_hbm, dst_hbm, asad_hbm, z_hbm, dp_hbm, agg_hbm,
               src_v, dst_v, as_v, ad_v, den_v, rowA_v, rowB_v, zb_v,
               diA_v, diB_v, agg_sh):
    c = lax.axis_index("c")
    s = lax.axis_index("s")

    pltpu.sync_copy(src_hbm.at[s], src_v)
    pltpu.sync_copy(dst_hbm.at[s], dst_v)
    pltpu.sync_copy(asad_hbm.at[0], as_v)
    pltpu.sync_copy(asad_hbm.at[1], ad_v)

    def xform(t, _):
        src_v[t, :] = src_v[t, :] * 2 + c
        return 0
    lax.fori_loop(0, CT, xform, 0)

    neg = jnp.full((16,), -3.0e38, jnp.float32)
    zero16 = jnp.zeros((16,), jnp.float32)

    def zmax_body(i, carry):
        ma, md = carry
        sl = pl.ds(i * 16, 16)
        den_v[sl] = zero16
        ma = jnp.maximum(ma, as_v[sl])
        md = jnp.maximum(md, ad_v[sl])
        return ma, md

    ma, md = lax.fori_loop(0, NP // 16, zmax_body, (neg, neg))

    iota16 = lax.iota(jnp.int32, 16)
    def _allmax(v):
        for sh in (1, 2, 4, 8):
            v = jnp.maximum(v, _permute(v, jnp.bitwise_xor(iota16, sh)))
        return v

    G = _allmax(ma) + _allmax(md)

    rows_per_tile = NP // 16
    def zb_zero(i, _):
        zb_v[i % 64, pl.ds((i // 64) * 16, 16)] = zero16
        return 0
    lax.fori_loop(0, 64 * 4, zb_zero, 0)

    def agg_zero(j, _):
        pltpu.sync_copy(zb_v, agg_sh.at[pl.ds(s * rows_per_tile + j * 64, 64)])
        return 0
    lax.fori_loop(0, rows_per_tile // 64, agg_zero, 0)

    plsc.subcore_barrier()

    def gather_scale(t, buf, dibuf):
        srcr = src_v[t, :]
        src16 = lax.shift_right_logical(srcr, 1)
        dst16 = dst_v[t, :]
        a_s = plsc.load_gather(as_v, [src16])
        a_d = plsc.load_gather(ad_v, [dst16])
        al = a_s + a_d
        al = jnp.where(al >= 0.0, al, al * 0.2)
        w = jnp.exp(al - G)
        plsc.addupdate_scatter(den_v, [dst16], w)
        dibuf[...] = dst16
        pltpu.sync_copy(z_hbm.at[src_v.at[t]], buf)
        for i in range(B):
            wb = _permute(w, jnp.full((16,), i, jnp.int32))
            for j in range(4):
                sl = pl.ds(j * 16, 16)
                buf[i, sl] = buf[i, sl] * wb

    gather_scale(0, rowA_v, diA_v)

    def pair(u, _):
        gather_scale(2 * u + 1, rowB_v, diB_v)
        pltpu.sync_copy(rowA_v, agg_sh.at[diA_v], add=True)

        @pl.when(2 * u + 2 < CT)
        def _():
            gather_scale(2 * u + 2, rowA_v, diA_v)
        pltpu.sync_copy(rowB_v, agg_sh.at[diB_v], add=True)
        return 0

    lax.fori_loop(0, CT // 2, pair, 0)

    plsc.subcore_barrier()

    @pl.when(c == 0)
    def _():
        pltpu.sync_copy(den_v, dp_hbm.at[s])

    def agg_out(j, _):
        r0 = s * rows_per_tile + j * 64
        pltpu.sync_copy(agg_sh.at[pl.ds(r0, 64)], zb_v)
        pltpu.sync_copy(zb_v, agg_hbm.at[pl.ds(r0, 64), c])
        return 0
    lax.fori_loop(0, rows_per_tile // 64, agg_out, 0)


@jax.jit
def _edge_pass(srcg, dstg, asad, z):
    mesh = plsc.VectorSubcoreMesh(core_axis_name="c", subcore_axis_name="s",
                                  num_cores=2, num_subcores=16)
    f = pl.kernel(
        _edge_body,
        out_type=(
            jax.ShapeDtypeStruct((NCH, NP), jnp.float32),
            jax.ShapeDtypeStruct((NP, 2, 64), jnp.float32),
        ),
        mesh=mesh,
        compiler_params=pltpu.CompilerParams(needs_layout_passes=False,
                                             use_tc_tiling_on_sc=False),
        scratch_types=[
            pltpu.VMEM((CT, B), jnp.int32),
            pltpu.VMEM((CT, B), jnp.int32),
            pltpu.VMEM((NP,), jnp.float32),
            pltpu.VMEM((NP,), jnp.float32),
            pltpu.VMEM((NP,), jnp.float32),
            pltpu.VMEM((B, 64), jnp.float32),
            pltpu.VMEM((B, 64), jnp.float32),
            pltpu.VMEM((64, 64), jnp.float32),
            pltpu.VMEM((B,), jnp.int32),
            pltpu.VMEM((B,), jnp.int32),
            pltpu.VMEM_SHARED((NP, 64), jnp.float32),
        ],
    )
    return f(srcg, dstg, asad, z.reshape(2 * NP, 64))


def _prep_body(ws_ref, wd_ref, as_ref, ad_ref, o_ref):
    r0 = lax.dot_general(as_ref[...], ws_ref[...], (((1,), (1,)), ((), ())),
                         preferred_element_type=jnp.float32)
    r1 = lax.dot_general(ad_ref[...], wd_ref[...], (((1,), (1,)), ((), ())),
                         preferred_element_type=jnp.float32)
    o_ref[...] = jnp.concatenate(
        [r0, r1, jnp.zeros((6, 128), jnp.float32)], axis=0)


def _prep(Ws, Wd, a_s, a_d):
    H = Ws.shape[1]
    return pl.pallas_call(
        _prep_body,
        out_shape=jax.ShapeDtypeStruct((8, 128), jnp.float32),
    )(Ws, Wd, a_s.reshape(1, H), a_d.reshape(1, H))


def _asad_body(x_ref, v_ref, o_ref):
    o_ref[...] = lax.dot_general(
        v_ref[...], x_ref[...], (((1,), (1,)), ((), ())),
        preferred_element_type=jnp.float32)


def _asad(xp, v2, bm=1024):
    return pl.pallas_call(
        _asad_body,
        grid=(NP // bm,),
        in_specs=[
            pl.BlockSpec((bm, 128), lambda m: (m, 0)),
            pl.BlockSpec((8, 128), lambda m: (0, 0)),
        ],
        out_specs=pl.BlockSpec((8, bm), lambda m: (0, m)),
        out_shape=jax.ShapeDtypeStruct((8, NP), jnp.float32),
    )(xp, v2)


def _k3_body(ap_ref, dp_ref, w_ref, b_ref, o_ref):
    a = ap_ref[...]
    dn = jnp.sum(dp_ref[...], axis=0)
    z = a * (1.0 / (dn + 1e-16))[:, None]
    y = jnp.dot(z, w_ref[...], preferred_element_type=jnp.float32) + b_ref[...]
    o_ref[...] = jnp.maximum(y, 0.0)


def _k3(agg_parts, denom_parts, W, b, bm=512):
    H = W.shape[1]
    return pl.pallas_call(
        _k3_body,
        grid=(NP // bm,),
        in_specs=[
            pl.BlockSpec((bm, 128), lambda m: (m, 0)),
            pl.BlockSpec((NCH, bm), lambda m: (0, m)),
            pl.BlockSpec((128, H), lambda m: (0, 0)),
            pl.BlockSpec((1, H), lambda m: (0, 0)),
        ],
        out_specs=pl.BlockSpec((bm, H), lambda m: (m, 0)),
        out_shape=jax.ShapeDtypeStruct((NP, H), jnp.float32),
    )(agg_parts, denom_parts, W, b.reshape(1, H))


def _mm_body(a_ref, w_ref, b_ref, o_ref):
    k = pl.program_id(2)

    @pl.when(k == 0)
    def _():
        o_ref[...] = jnp.broadcast_to(b_ref[...], o_ref.shape)

    o_ref[...] += jnp.dot(a_ref[...], w_ref[...],
                          preferred_element_type=jnp.float32)


def _mm(a, w, b, bm=512, bn=512, bk=512):
    M, K = a.shape
    _, Nf = w.shape
    bn = min(bn, Nf)
    bk = min(bk, K)
    nk = K // bk
    return pl.pallas_call(
        _mm_body,
        grid=(M // bm, Nf // bn, nk),
        in_specs=[
            pl.BlockSpec((bm, bk), lambda m, n, k: (m, k)),
            pl.BlockSpec((bk, bn), lambda m, n, k: (k, n)),
            pl.BlockSpec((1, bn), lambda m, n, k: (0, n)),
        ],
        out_specs=pl.BlockSpec((bm, bn), lambda m, n, k: (m, n)),
        out_shape=jax.ShapeDtypeStruct((M, Nf), jnp.float32),
        compiler_params=pltpu.CompilerParams(
            dimension_semantics=("parallel", "parallel", "arbitrary")),
    )(a, w, b.reshape(1, Nf))


def _stats_body(y_ref, s_ref, *, bm, nm):
    n_i, m_i = pl.program_id(0), pl.program_id(1)

    @pl.when(m_i == 0)
    def _():
        s_ref[...] = jnp.zeros_like(s_ref)

    rows = m_i * bm + lax.broadcasted_iota(jnp.int32, (bm, 1), 0)
    yb = jnp.where(rows < N, y_ref[...], 0.0)
    s0 = jnp.sum(yb, axis=0, keepdims=True)
    s1 = jnp.sum(yb * yb, axis=0, keepdims=True)
    bn = s0.shape[1]
    s_ref[...] += jnp.concatenate(
        [s0, s1, jnp.zeros((6, bn), jnp.float32)], axis=0)


def _stats(y, bm=1024, bn=1024):
    M, F = y.shape
    bn = min(bn, F)
    nm = M // bm
    return pl.pallas_call(
        functools.partial(_stats_body, bm=bm, nm=nm),
        grid=(F // bn, nm),
        in_specs=[pl.BlockSpec((bm, bn), lambda n, m: (m, n))],
        out_specs=pl.BlockSpec((8, bn), lambda n, m: (0, n)),
        out_shape=jax.ShapeDtypeStruct((8, F), jnp.float32),
        compiler_params=pltpu.CompilerParams(
            dimension_semantics=("parallel", "arbitrary")),
    )(y)


def _bn_body(y_ref, s_ref, g_ref, b_ref, o_ref):
    mean = s_ref[0:1, :] * (1.0 / N)
    var = s_ref[1:2, :] * (1.0 / N) - mean * mean
    scale = g_ref[...] * lax.rsqrt(var + EPS)
    o_ref[...] = jnp.maximum((y_ref[...] - mean) * scale + b_ref[...], 0.0)


def _bn(y, s, g, b, bm=1024, bn=1024):
    M, F = y.shape
    bn = min(bn, F)
    return pl.pallas_call(
        _bn_body,
        grid=(M // bm, F // bn),
        in_specs=[
            pl.BlockSpec((bm, bn), lambda m, n: (m, n)),
            pl.BlockSpec((8, bn), lambda m, n: (0, n)),
            pl.BlockSpec((1, bn), lambda m, n: (0, n)),
            pl.BlockSpec((1, bn), lambda m, n: (0, n)),
        ],
        out_specs=pl.BlockSpec((bm, bn), lambda m, n: (m, n)),
        out_shape=jax.ShapeDtypeStruct((M, F), jnp.float32),
    )(y, s, g.reshape(1, F), b.reshape(1, F))


def _lsm_body(z_ref, o_ref):
    z = z_ref[...]
    mx = jnp.max(z, axis=1, keepdims=True)
    e = jnp.exp(z - mx)
    lse = jnp.log(jnp.sum(e, axis=1, keepdims=True))
    o_ref[...] = z - mx - lse


def _lsm(z, bm=1024):
    M, F = z.shape
    return pl.pallas_call(
        _lsm_body,
        grid=(M // bm,),
        in_specs=[pl.BlockSpec((bm, F), lambda m: (m, 0))],
        out_specs=pl.BlockSpec((bm, F), lambda m: (m, 0)),
        out_shape=jax.ShapeDtypeStruct((M, F), jnp.float32),
    )(z)


def _gat_layer(zp, srcg, dstg, Ws, Wd, a_s, a_d, b):
    v2 = _prep(Ws, Wd, a_s, a_d)
    asad8 = _asad(zp, v2)
    denom_parts, agg3 = _edge_pass(srcg, dstg, asad8[:2], zp)
    return _k3(agg3.reshape(NP, 128), denom_parts, Ws, b)


def kernel(x, edge_index, W_src1, W_dst1, att_src1, att_dst1, bg1,
           W_src2, W_dst2, att_src2, att_dst2, bg2,
           Wf1, bf1, g1, be1, Wf2, bf2, g2, be2, Wf3, bf3, g3, be3,
           Wf4, bf4):
    xp = jnp.pad(x, ((0, NP - N), (0, 0)))
    loop = jnp.arange(N, dtype=edge_index.dtype)
    pad = jnp.full((EP - ET,), N, edge_index.dtype)
    srcg = jnp.concatenate([edge_index[0], loop, pad]).reshape(NCH, CT, B)
    dstg = jnp.concatenate([edge_index[1], loop, pad]).reshape(NCH, CT, B)

    h = _gat_layer(xp, srcg, dstg, W_src1, W_dst1, att_src1, att_dst1, bg1)
    h = _gat_layer(h, srcg, dstg, W_src2, W_dst2, att_src2, att_dst2, bg2)

    F1p = 4096
    Wf1p = jnp.pad(Wf1, ((0, 0), (0, F1p - Wf1.shape[1])))
    bf1p = jnp.pad(bf1, (0, F1p - bf1.shape[0]))
    g1p = jnp.pad(g1, (0, F1p - g1.shape[0]))
    be1p = jnp.pad(be1, (0, F1p - be1.shape[0]))
    Wf2p = jnp.pad(Wf2, ((0, F1p - Wf2.shape[0]), (0, 0)))

    y = _mm(h, Wf1p, bf1p)
    y = _bn(y, _stats(y), g1p, be1p)
    y = _mm(y, Wf2p, bf2)
    y = _bn(y, _stats(y), g2, be2)
    y = _mm(y, Wf3, bf3)
    y = _bn(y, _stats(y), g3, be3)

    OUTP = 128
    Wf4p = jnp.pad(Wf4, ((0, 0), (0, OUTP - Wf4.shape[1])))
    bf4p = jnp.pad(bf4, (0, OUTP - bf4.shape[0]), constant_values=-1e9)
    y = _mm(y, Wf4p, bf4p, bn=128, bk=256)
    out = _lsm(y)
    return out[:N, :Wf4.shape[1]]
